# baseline (speedup 1.0000x reference)
import functools

import jax
import jax.numpy as jnp
from jax import lax
from jax.experimental import pallas as pl
from jax.experimental.pallas import tpu as pltpu
from jax.experimental.pallas import tpu_sc as plsc
from jax.experimental import layout

_VOCAB = 1000
_D1 = 32
_D2 = 16
_B = 16384
_R1 = _D1 * _D1
_R2 = _D2 * _D2

_NC = 2
_NS = 16
_NW = _NC * _NS
_BPW = _B // _NW
_C1 = 32
_C2 = 64
_N1 = _BPW // _C1
_N2 = _BPW // _C2


def _sc_gather(x, t1, t2):
  mesh = plsc.VectorSubcoreMesh(core_axis_name="c", subcore_axis_name="s")

  @functools.partial(
      pl.kernel,
      out_type=(
          jax.ShapeDtypeStruct((_B, _R1), jnp.float32),
          jax.ShapeDtypeStruct((_B, _R2), jnp.float32),
      ),
      mesh=mesh,
      scratch_types=[
          pltpu.VMEM((_BPW,), jnp.int32),
          pltpu.VMEM((2, _C1, _R1), jnp.float32),
          pltpu.VMEM((2, _C2, _R2), jnp.float32),
          pltpu.SemaphoreType.DMA,
          pltpu.SemaphoreType.DMA,
      ],
  )
  def body(x_hbm, t1_hbm, t2_hbm, o1_hbm, o2_hbm, idx_v, buf1, buf2, g0, g1):
    wid = lax.axis_index("s") * _NC + lax.axis_index("c")
    base = wid * _BPW
    gsem = (g0, g1)
    pltpu.sync_copy(x_hbm.at[pl.ds(base, _BPW)], idx_v)

    def fire1(c):
      p = c & 1
      i = idx_v.at[pl.ds(c * _C1, _C1)]
      return pltpu.async_copy(t1_hbm.at[i], buf1.at[p], gsem[p])

    def fire2(c):
      p = c & 1
      i = idx_v.at[pl.ds(c * _C2, _C2)]
      return pltpu.async_copy(t2_hbm.at[i], buf2.at[p], gsem[p])

    pending = fire1(0)
    for c in range(_N1):
      p = c & 1
      nxt = fire1(c + 1) if c + 1 < _N1 else fire2(0)
      pending.wait()
      pltpu.sync_copy(buf1.at[p], o1_hbm.at[pl.ds(base + c * _C1, _C1)])
      pending = nxt

    for c in range(_N2):
      p = c & 1
      nxt = fire2(c + 1) if c + 1 < _N2 else None
      pending.wait()
      pltpu.sync_copy(buf2.at[p], o2_hbm.at[pl.ds(base + c * _C2, _C2)])
      pending = nxt

  return body(x, t1, t2)


def _impl(x, T1, T2):
  xi = x.astype(jnp.int32)
  t1 = T1.reshape(_VOCAB, _R1)
  t2 = T2.reshape(_VOCAB, _R2)
  o1, o2 = _sc_gather(xi, t1, t2)
  return (o1.reshape(_B * _D1, _D1), o2.reshape(_B * _D2, _D2))


_jitted = []


def kernel(x, T1, T2):
  if not _jitted:
    sh = jax.sharding.SingleDeviceSharding(jax.devices()[0])
    lin = lambda: layout.Format(
        layout.Layout(major_to_minor=(0, 1), tiling=()), sh)
    _jitted.append(jax.jit(_impl, out_shardings=(lin(), lin())))
  return _jitted[0](x, T1, T2)


# speedup vs baseline: 1.1449x; 1.1449x over previous
import functools

import jax
import jax.numpy as jnp
from jax import lax
from jax.experimental import pallas as pl
from jax.experimental.pallas import tpu as pltpu
from jax.experimental.pallas import tpu_sc as plsc
from jax.experimental import layout

_VOCAB = 1000
_D1 = 32
_D2 = 16
_B = 16384
_R1 = _D1 * _D1
_R2 = _D2 * _D2

_NC = 2
_NS = 16
_NW = _NC * _NS
_BPW = _B // _NW
_C = 128
_NCHUNK = _BPW // _C

_BB = 1024


def _sc_gather_t2(x, t2):
  mesh = plsc.VectorSubcoreMesh(core_axis_name="c", subcore_axis_name="s")

  @functools.partial(
      pl.kernel,
      out_type=jax.ShapeDtypeStruct((_B, _R2), jnp.float32),
      mesh=mesh,
      scratch_types=[
          pltpu.VMEM((_BPW,), jnp.int32),
          pltpu.VMEM((2, _C, _R2), jnp.float32),
          pltpu.SemaphoreType.DMA,
          pltpu.SemaphoreType.DMA,
      ],
  )
  def body(x_hbm, t2_hbm, o2_hbm, idx_v, buf2, g0, g1):
    wid = lax.axis_index("s") * _NC + lax.axis_index("c")
    base = wid * _BPW
    gsem = (g0, g1)
    pltpu.sync_copy(x_hbm.at[pl.ds(base, _BPW)], idx_v)

    def fire(c):
      p = c & 1
      i = idx_v.at[pl.ds(c * _C, _C)]
      return pltpu.async_copy(t2_hbm.at[i], buf2.at[p], gsem[p])

    pending = fire(0)
    for c in range(_NCHUNK):
      p = c & 1
      cb = base + c * _C
      nxt = fire(c + 1) if c + 1 < _NCHUNK else None
      pending.wait()
      pltpu.sync_copy(buf2.at[p], o2_hbm.at[pl.ds(cb, _C)])
      pending = nxt

  return body(x, t2)


def _tc_gather_body(xs_ref, t1_ref, o1_ref):
  base = pl.program_id(0) * _BB

  def body(j, carry):
    i = xs_ref[base + j]
    o1_ref[pl.ds(j, 1)] = t1_ref[pl.ds(i, 1)]
    return carry

  lax.fori_loop(0, _BB, body, 0, unroll=8)


def _tc_gather_t1(x, t1):
  grid_spec = pltpu.PrefetchScalarGridSpec(
      num_scalar_prefetch=1,
      grid=(_B // _BB,),
      in_specs=[
          pl.BlockSpec((_VOCAB, 8, 128), lambda step, xs: (0, 0, 0)),
      ],
      out_specs=pl.BlockSpec((_BB, 8, 128), lambda step, xs: (step, 0, 0)),
  )
  return pl.pallas_call(
      _tc_gather_body,
      grid_spec=grid_spec,
      out_shape=jax.ShapeDtypeStruct((_B, 8, 128), jnp.float32),
  )(x, t1)


def _impl(x, T1, T2):
  xi = x.astype(jnp.int32)
  t1 = T1.reshape(_VOCAB, 8, 128)
  t2 = T2.reshape(_VOCAB, _R2)
  o2 = _sc_gather_t2(xi, t2)
  o1 = _tc_gather_t1(xi, t1)
  return (o1.reshape(_B * _D1, _D1), o2.reshape(_B * _D2, _D2))


_jitted = []


def kernel(x, T1, T2):
  if not _jitted:
    sh = jax.sharding.SingleDeviceSharding(jax.devices()[0])
    lin2 = layout.Format(layout.Layout(major_to_minor=(0, 1), tiling=()), sh)
    _jitted.append(jax.jit(_impl, out_shardings=(layout.Format(None, sh), lin2)))
  return _jitted[0](x, T1, T2)
